# baseline (speedup 1.0000x reference)
import jax
import jax.numpy as jnp
from jax import lax
from jax.experimental import pallas as pl
from jax.experimental.pallas import tpu as pltpu
from jax.scipy.special import gammaln

B, L, D, H = 16, 512, 768, 2048
TEMP = 1.0
PRIOR = 0.2
THRESHOLD = 0.5
RG = 2
NSTEPS = B // RG
HC = 512
NHC = H // HC


def _body(x_ref, u_ref, am_ref, w1_ref, b1_ref, w2_ref, b2_ref,
          pooled_ref, nb_ref, tp_ref, sam_ref):
    step = pl.program_id(0)

    xg = x_ref[0]
    h = jnp.maximum(jnp.dot(xg, w1_ref[...], preferred_element_type=jnp.float32)
                    + b1_ref[...], 0.0)
    logits = jnp.dot(h, w2_ref[...], preferred_element_type=jnp.float32) \
        + b2_ref[...]

    ug = u_ref[0]
    amg = am_ref[0]
    noise = jnp.log(ug) - jnp.log1p(-ug)
    soft = jax.nn.sigmoid((logits + noise) / TEMP)
    hard = (soft > THRESHOLD).astype(jnp.float32) * amg

    iota_k = lax.broadcasted_iota(jnp.int32, (L, L), 0)
    iota_l = lax.broadcasted_iota(jnp.int32, (L, L), 1)
    m_strict = (iota_k < iota_l).astype(jnp.float32)
    iota_s = lax.broadcasted_iota(jnp.int32, (L, L), 0).astype(jnp.float32)

    nb_acc = jnp.zeros((1, 1), jnp.float32)
    for r in range(RG):
        xr = xg[r * L:(r + 1) * L, :]
        hr = hard[r * L:(r + 1) * L, :]
        seg_row = lax.dot_general(hr, m_strict, (((0,), (0,)), ((), ())),
                                  preferred_element_type=jnp.float32)
        mask_t = (iota_s == seg_row).astype(jnp.float32)
        counts = jnp.sum(mask_t, axis=1, keepdims=True)
        bar_t = mask_t * (1.0 / (counts + 1e-9))
        pooled_ref[0, r * L:(r + 1) * L, :] = \
            jnp.dot(bar_t, xr, preferred_element_type=jnp.float32)

        cb11 = jnp.sum(hr, axis=(0, 1), keepdims=True)
        iota_row = lax.broadcasted_iota(jnp.int32, (1, L), 1).astype(jnp.float32)
        sam_ref[0, :, r * L:(r + 1) * L] = (iota_row < cb11).astype(jnp.float32)
        nb_acc = nb_acc + cb11

    @pl.when(step == 0)
    def _():
        nb_ref[...] = jnp.zeros((1, 1), jnp.float32)
        tp_ref[...] = jnp.zeros((1, 1), jnp.float32)

    nb_ref[...] += nb_acc
    tp_ref[...] += jnp.sum(amg, axis=(0, 1), keepdims=True)


def _launch(hidden4, u4, am4, W1, b1r, W2, b2r):
    return pl.pallas_call(
        _body,
        grid=(NSTEPS,),
        in_specs=[
            pl.BlockSpec((1, RG * L, D), lambda b: (b, 0, 0)),
            pl.BlockSpec((1, RG * L, 1), lambda b: (b, 0, 0)),
            pl.BlockSpec((1, RG * L, 1), lambda b: (b, 0, 0)),
            pl.BlockSpec((D, H), lambda b: (0, 0)),
            pl.BlockSpec((1, H), lambda b: (0, 0)),
            pl.BlockSpec((H, 1), lambda b: (0, 0)),
            pl.BlockSpec((1, 1), lambda b: (0, 0)),
        ],
        out_specs=[
            pl.BlockSpec((1, RG * L, D), lambda b: (b, 0, 0)),
            pl.BlockSpec((1, 1), lambda b: (0, 0)),
            pl.BlockSpec((1, 1), lambda b: (0, 0)),
            pl.BlockSpec((1, 1, RG * L), lambda b: (b, 0, 0)),
        ],
        out_shape=[
            jax.ShapeDtypeStruct((NSTEPS, RG * L, D), jnp.float32),
            jax.ShapeDtypeStruct((1, 1), jnp.float32),
            jax.ShapeDtypeStruct((1, 1), jnp.float32),
            jax.ShapeDtypeStruct((NSTEPS, 1, RG * L), jnp.float32),
        ],
        compiler_params=pltpu.CompilerParams(
            dimension_semantics=("arbitrary",),
        ),
    )(hidden4, u4, am4, W1, b1r, W2, b2r)


def kernel(hidden, attention_mask, W1, b1, W2, b2, u):
    hidden4 = hidden.reshape(NSTEPS, RG * L, D)
    u4 = u.reshape(NSTEPS, RG * L, 1)
    am4 = attention_mask.reshape(NSTEPS, RG * L, 1)
    b1r = b1[None, :]
    b2r = b2[None, :]
    pooled4, nb, tp, sam4 = _launch(hidden4, u4, am4, W1, b1r, W2, b2r)
    pooled = pooled4.reshape(B, L, D)
    shortened_attention_mask = sam4.reshape(B, L)
    num_boundaries = nb[0, 0]
    total_positions = tp[0, 0]
    k, n = num_boundaries, total_positions
    logp = (gammaln(n + 1.0) - gammaln(k + 1.0) - gammaln(n - k + 1.0)
            + k * jnp.log(PRIOR) + (n - k) * jnp.log1p(-PRIOR))
    loss = -logp / n
    return (pooled, loss, num_boundaries, total_positions,
            shortened_attention_mask)


# speedup vs baseline: 1.0689x; 1.0689x over previous
import jax
import jax.numpy as jnp
from jax import lax
from jax.experimental import pallas as pl
from jax.experimental.pallas import tpu as pltpu
from jax.scipy.special import gammaln

B, L, D, H = 16, 512, 768, 2048
TEMP = 1.0
PRIOR = 0.2
THRESHOLD = 0.5
RG = 2
NSTEPS = B // RG


def _body(x_ref, u_ref, am_ref, w1_ref, b1_ref, w2_ref, b2_ref,
          pooled_ref, nb_ref, tp_ref, sam_ref):
    step = pl.program_id(0)

    xg = x_ref[...].reshape(RG * L, D)
    h = jnp.maximum(jnp.dot(xg, w1_ref[...], preferred_element_type=jnp.float32)
                    + b1_ref[...], 0.0)
    logits = jnp.dot(h, w2_ref[...], preferred_element_type=jnp.float32) \
        + b2_ref[...]

    ug = u_ref[...].reshape(RG * L, 1)
    amg = am_ref[...].reshape(RG * L, 1)
    noise = jnp.log(ug) - jnp.log1p(-ug)
    soft = jax.nn.sigmoid((logits + noise) / TEMP)
    hard = (soft > THRESHOLD).astype(jnp.float32) * amg

    iota_k = lax.broadcasted_iota(jnp.int32, (L, L), 0)
    iota_l = lax.broadcasted_iota(jnp.int32, (L, L), 1)
    m_strict = (iota_k < iota_l).astype(jnp.float32)
    iota_s = lax.broadcasted_iota(jnp.int32, (L, L), 0).astype(jnp.float32)

    nb_acc = jnp.zeros((1, 1), jnp.float32)
    for r in range(RG):
        xr = xg[r * L:(r + 1) * L, :]
        hr = hard[r * L:(r + 1) * L, :]
        seg_row = lax.dot_general(hr, m_strict, (((0,), (0,)), ((), ())),
                                  preferred_element_type=jnp.float32)
        mask_t = (iota_s == seg_row).astype(jnp.float32)
        counts = jnp.sum(mask_t, axis=1, keepdims=True)
        bar_t = mask_t * (1.0 / (counts + 1e-9))
        pooled_ref[r] = jnp.dot(bar_t, xr, preferred_element_type=jnp.float32)

        cb11 = jnp.sum(hr, axis=(0, 1), keepdims=True)
        iota_row = lax.broadcasted_iota(jnp.int32, (1, L), 1).astype(jnp.float32)
        sam_ref[r] = (iota_row < cb11).astype(jnp.float32)
        nb_acc = nb_acc + cb11

    @pl.when(step == 0)
    def _():
        nb_ref[...] = jnp.zeros((1, 1), jnp.float32)
        tp_ref[...] = jnp.zeros((1, 1), jnp.float32)

    nb_ref[...] += nb_acc
    tp_ref[...] += jnp.sum(am_ref[...], axis=(0, 1, 2), keepdims=True)[0]


def _launch(hidden, u3, am3, W1, b1r, W2, b2r):
    return pl.pallas_call(
        _body,
        grid=(NSTEPS,),
        in_specs=[
            pl.BlockSpec((RG, L, D), lambda b: (b, 0, 0)),
            pl.BlockSpec((RG, L, 1), lambda b: (b, 0, 0)),
            pl.BlockSpec((RG, L, 1), lambda b: (b, 0, 0)),
            pl.BlockSpec((D, H), lambda b: (0, 0)),
            pl.BlockSpec((1, H), lambda b: (0, 0)),
            pl.BlockSpec((H, 1), lambda b: (0, 0)),
            pl.BlockSpec((1, 1), lambda b: (0, 0)),
        ],
        out_specs=[
            pl.BlockSpec((RG, L, D), lambda b: (b, 0, 0)),
            pl.BlockSpec((1, 1), lambda b: (0, 0)),
            pl.BlockSpec((1, 1), lambda b: (0, 0)),
            pl.BlockSpec((RG, 1, L), lambda b: (b, 0, 0)),
        ],
        out_shape=[
            jax.ShapeDtypeStruct((B, L, D), jnp.float32),
            jax.ShapeDtypeStruct((1, 1), jnp.float32),
            jax.ShapeDtypeStruct((1, 1), jnp.float32),
            jax.ShapeDtypeStruct((B, 1, L), jnp.float32),
        ],
        compiler_params=pltpu.CompilerParams(
            dimension_semantics=("arbitrary",),
        ),
    )(hidden, u3, am3, W1, b1r, W2, b2r)


def kernel(hidden, attention_mask, W1, b1, W2, b2, u):
    u3 = u[:, :, None]
    am3 = attention_mask[:, :, None]
    b1r = b1[None, :]
    b2r = b2[None, :]
    pooled, nb, tp, sam = _launch(hidden, u3, am3, W1, b1r, W2, b2r)
    num_boundaries = nb[0, 0]
    total_positions = tp[0, 0]
    shortened_attention_mask = sam[:, 0, :]
    k, n = num_boundaries, total_positions
    logp = (gammaln(n + 1.0) - gammaln(k + 1.0) - gammaln(n - k + 1.0)
            + k * jnp.log(PRIOR) + (n - k) * jnp.log1p(-PRIOR))
    loss = -logp / n
    return (pooled, loss, num_boundaries, total_positions,
            shortened_attention_mask)


# speedup vs baseline: 1.1068x; 1.0355x over previous
import jax
import jax.numpy as jnp
from jax import lax
from jax.experimental import pallas as pl
from jax.experimental.pallas import tpu as pltpu
from jax.scipy.special import gammaln

B, L, D, H = 16, 512, 768, 2048
TEMP = 1.0
PRIOR = 0.2
THRESHOLD = 0.5
RG = 2
NSTEPS = B // RG


def _body(x_ref, u_ref, am_ref, w1_ref, b1_ref, w2_ref, b2_ref,
          pooled_ref, nb_ref, tp_ref, sam_ref):
    step = pl.program_id(0)

    xg = x_ref[...].reshape(RG * L, D)
    h = jnp.maximum(jnp.dot(xg, w1_ref[...], preferred_element_type=jnp.float32)
                    + b1_ref[...], 0.0)
    ug = u_ref[...].reshape(RG * L, 1)
    amg = am_ref[...].reshape(RG * L, 1)
    noise = jnp.log(ug) - jnp.log1p(-ug)
    hards = []
    for r in range(RG):
        lg_r = jnp.dot(h[r * L:(r + 1) * L, :], w2_ref[...],
                       preferred_element_type=jnp.float32) + b2_ref[...]
        soft_r = jax.nn.sigmoid((lg_r + noise[r * L:(r + 1) * L, :]) / TEMP)
        hards.append((soft_r > THRESHOLD).astype(jnp.float32)
                     * amg[r * L:(r + 1) * L, :])
    hard = jnp.concatenate(hards, axis=0)

    iota_k = lax.broadcasted_iota(jnp.int32, (L, L), 0)
    iota_l = lax.broadcasted_iota(jnp.int32, (L, L), 1)
    m_strict = (iota_k < iota_l).astype(jnp.float32)
    iota_s = lax.broadcasted_iota(jnp.int32, (L, L), 0).astype(jnp.float32)

    nb_acc = jnp.zeros((1, 1), jnp.float32)
    bars = []
    for r in range(RG):
        xr = xg[r * L:(r + 1) * L, :]
        hr = hard[r * L:(r + 1) * L, :]
        seg_row = lax.dot_general(hr, m_strict, (((0,), (0,)), ((), ())),
                                  preferred_element_type=jnp.float32)
        mask_t = (iota_s == seg_row).astype(jnp.float32)
        counts = jnp.sum(mask_t, axis=1, keepdims=True)
        bar_t = mask_t * (1.0 / (counts + 1e-9))
        bars.append(bar_t)

        cb11 = jnp.sum(hr, axis=(0, 1), keepdims=True)
        iota_row = lax.broadcasted_iota(jnp.int32, (1, L), 1).astype(jnp.float32)
        sam_ref[r] = (iota_row < cb11).astype(jnp.float32)
        nb_acc = nb_acc + cb11

    for r in range(RG):
        pooled_ref[r] = jnp.dot(bars[r], xg[r * L:(r + 1) * L, :],
                                preferred_element_type=jnp.float32)

    @pl.when(step == 0)
    def _():
        nb_ref[...] = jnp.zeros((1, 1), jnp.float32)
        tp_ref[...] = jnp.zeros((1, 1), jnp.float32)

    nb_ref[...] += nb_acc
    tp_ref[...] += jnp.sum(am_ref[...], axis=(0, 1, 2), keepdims=True)[0]


def _launch(hidden, u3, am3, W1, b1r, W2, b2r):
    return pl.pallas_call(
        _body,
        grid=(NSTEPS,),
        in_specs=[
            pl.BlockSpec((RG, L, D), lambda b: (b, 0, 0)),
            pl.BlockSpec((RG, L, 1), lambda b: (b, 0, 0)),
            pl.BlockSpec((RG, L, 1), lambda b: (b, 0, 0)),
            pl.BlockSpec((D, H), lambda b: (0, 0)),
            pl.BlockSpec((1, H), lambda b: (0, 0)),
            pl.BlockSpec((H, 1), lambda b: (0, 0)),
            pl.BlockSpec((1, 1), lambda b: (0, 0)),
        ],
        out_specs=[
            pl.BlockSpec((RG, L, D), lambda b: (b, 0, 0)),
            pl.BlockSpec((1, 1), lambda b: (0, 0)),
            pl.BlockSpec((1, 1), lambda b: (0, 0)),
            pl.BlockSpec((RG, 1, L), lambda b: (b, 0, 0)),
        ],
        out_shape=[
            jax.ShapeDtypeStruct((B, L, D), jnp.float32),
            jax.ShapeDtypeStruct((1, 1), jnp.float32),
            jax.ShapeDtypeStruct((1, 1), jnp.float32),
            jax.ShapeDtypeStruct((B, 1, L), jnp.float32),
        ],
        compiler_params=pltpu.CompilerParams(
            dimension_semantics=("arbitrary",),
        ),
    )(hidden, u3, am3, W1, b1r, W2, b2r)


def kernel(hidden, attention_mask, W1, b1, W2, b2, u):
    u3 = u[:, :, None]
    am3 = attention_mask[:, :, None]
    b1r = b1[None, :]
    b2r = b2[None, :]
    pooled, nb, tp, sam = _launch(hidden, u3, am3, W1, b1r, W2, b2r)
    num_boundaries = nb[0, 0]
    total_positions = tp[0, 0]
    shortened_attention_mask = sam[:, 0, :]
    k, n = num_boundaries, total_positions
    logp = (gammaln(n + 1.0) - gammaln(k + 1.0) - gammaln(n - k + 1.0)
            + k * jnp.log(PRIOR) + (n - k) * jnp.log1p(-PRIOR))
    loss = -logp / n
    return (pooled, loss, num_boundaries, total_positions,
            shortened_attention_mask)
